# baseline (speedup 1.0000x reference)
import functools

import jax
import jax.numpy as jnp
from jax import lax
from jax.experimental import pallas as pl
from jax.experimental.pallas import tpu as pltpu
from jax.experimental.pallas import tpu_sc as plsc

BATCH = 4096
HIST_LEN = 200
EMB = 32

_NC = 2
_NS = 16
_NW = _NC * _NS

_BPW = BATCH // _NW
_K = 4
_BLK_ROWS = _K * _BPW
_NBLK = HIST_LEN // _K
_NPAIR = _NBLK // 2


def _gather_kernel(idx_hbm, table_hbm, out_hbm,
                   idx_v, rows_a, rows_b, tbuf_a, tbuf_b,
                   gsem_a, gsem_b, osem_a, osem_b):
    wid = lax.axis_index("s") * _NC + lax.axis_index("c")
    col = wid * _BPW
    pltpu.sync_copy(idx_hbm.at[:, pl.ds(col, _BPW)], idx_v)

    lanes = jnp.arange(16, dtype=jnp.int32)

    def fire(rows, gsem, blk):
        for k in range(_K):
            pltpu.async_copy(
                table_hbm.at[idx_v.at[blk * _K + k]],
                rows.at[pl.ds(k * _BPW, _BPW)],
                gsem,
            )

    def drain(rows, gsem):
        pltpu.make_async_copy(table_hbm.at[pl.ds(0, _BLK_ROWS)], rows, gsem
                              ).wait()

    def transpose(rows, tbuf):
        def tstep(g, carry):
            r0 = g * 16
            k = r0 // _BPW
            off = r0 - k * _BPW
            row_idx = lanes + r0
            for e in range(EMB):
                col_idx = jnp.full((16,), e, jnp.int32)
                v = plsc.load_gather(rows, [row_idx, col_idx])
                tbuf[k, e, pl.ds(off, 16)] = v
            return carry

        lax.fori_loop(0, _BLK_ROWS // 16, tstep, 0)

    fire(rows_a, gsem_a, 0)
    fire(rows_b, gsem_b, 1)

    def body(p, carry):
        blk_a = 2 * p
        blk_b = 2 * p + 1
        drain(rows_a, gsem_a)
        transpose(rows_a, tbuf_a)
        out_a = pltpu.async_copy(
            tbuf_a,
            out_hbm.at[pl.ds(blk_a * _K, _K), :, pl.ds(col, _BPW)],
            osem_a)
        fire(rows_a, gsem_a, lax.rem(blk_a + 2, _NBLK))
        drain(rows_b, gsem_b)
        transpose(rows_b, tbuf_b)
        out_b = pltpu.async_copy(
            tbuf_b,
            out_hbm.at[pl.ds(blk_b * _K, _K), :, pl.ds(col, _BPW)],
            osem_b)
        fire(rows_b, gsem_b, lax.rem(blk_b + 2, _NBLK))
        out_a.wait()
        out_b.wait()
        return carry

    lax.fori_loop(0, _NPAIR, body, 0)
    drain(rows_a, gsem_a)
    drain(rows_b, gsem_b)


@jax.jit
def _gather(idx_t, table):
    run = pl.kernel(
        _gather_kernel,
        out_type=jax.ShapeDtypeStruct((HIST_LEN, EMB, BATCH), jnp.float32),
        mesh=plsc.VectorSubcoreMesh(core_axis_name="c", subcore_axis_name="s"),
        scratch_types=[
            pltpu.VMEM((HIST_LEN, _BPW), jnp.int32),
            pltpu.VMEM((_BLK_ROWS, EMB), jnp.float32),
            pltpu.VMEM((_BLK_ROWS, EMB), jnp.float32),
            pltpu.VMEM((_K, EMB, _BPW), jnp.float32),
            pltpu.VMEM((_K, EMB, _BPW), jnp.float32),
            pltpu.SemaphoreType.DMA,
            pltpu.SemaphoreType.DMA,
            pltpu.SemaphoreType.DMA,
            pltpu.SemaphoreType.DMA,
        ],
        compiler_params=pltpu.CompilerParams(
            use_tc_tiling_on_sc=False, needs_layout_passes=False),
    )
    return run(idx_t, table)


def kernel(input, W):
    idx_t = jnp.transpose(input).astype(jnp.int32)
    out_t = _gather(idx_t, W)
    return jnp.expand_dims(jnp.transpose(out_t, (2, 0, 1)), -1)


# speedup vs baseline: 1.3765x; 1.3765x over previous
import functools

import jax
import jax.numpy as jnp
from jax import lax
from jax.experimental import pallas as pl
from jax.experimental.pallas import tpu as pltpu
from jax.experimental.pallas import tpu_sc as plsc

BATCH = 4096
HIST_LEN = 200
EMB = 32

_NC = 2
_NS = 16
_NW = _NC * _NS

_BPW = BATCH // _NW
_K = 4
_BLK_ROWS = _K * _BPW
_NBLK = HIST_LEN // _K
_NPAIR = _NBLK // 2


def _gather_kernel(idx_hbm, table_hbm, out_hbm,
                   idx_v, rows_a, rows_b, tbuf_a, tbuf_b,
                   gsem_a, gsem_b, osem_a, osem_b):
    wid = lax.axis_index("s") * _NC + lax.axis_index("c")
    col = wid * _BPW
    pltpu.sync_copy(idx_hbm.at[:, pl.ds(col, _BPW)], idx_v)

    lanes = jnp.arange(16, dtype=jnp.int32)

    def fire(rows, gsem, blk):
        for k in range(_K):
            pltpu.async_copy(
                table_hbm.at[idx_v.at[blk * _K + k]],
                rows.at[pl.ds(k * _BPW, _BPW)],
                gsem,
            )

    def drain(rows, gsem):
        pltpu.make_async_copy(table_hbm.at[pl.ds(0, _BLK_ROWS)], rows, gsem
                              ).wait()

    def transpose(rows, tbuf):
        def tstep(g, carry):
            r0 = g * 16
            k = g // 8
            off = (g % 8) * 16
            row_idx = lanes + r0
            for e0 in (0, 16):
                vs = [
                    plsc.load_gather(
                        rows,
                        [row_idx, jnp.full((16,), e0 + i, jnp.int32)])
                    for i in range(16)
                ]
                for i in range(16):
                    e = e0 + i
                    tbuf[k, e // 8, 0, e % 8, pl.ds(off, 16)] = vs[i]
            return carry

        lax.fori_loop(0, _BLK_ROWS // 16, tstep, 0)

    fire(rows_a, gsem_a, 0)
    fire(rows_b, gsem_b, 1)

    def body(p, carry):
        blk_a = 2 * p
        blk_b = 2 * p + 1
        drain(rows_a, gsem_a)
        transpose(rows_a, tbuf_a)
        out_a = pltpu.async_copy(
            tbuf_a,
            out_hbm.at[pl.ds(blk_a * _K, _K), :, pl.ds(wid, 1)],
            osem_a)
        fire(rows_a, gsem_a, lax.rem(blk_a + 2, _NBLK))
        drain(rows_b, gsem_b)
        transpose(rows_b, tbuf_b)
        out_b = pltpu.async_copy(
            tbuf_b,
            out_hbm.at[pl.ds(blk_b * _K, _K), :, pl.ds(wid, 1)],
            osem_b)
        fire(rows_b, gsem_b, lax.rem(blk_b + 2, _NBLK))
        out_a.wait()
        out_b.wait()
        return carry

    lax.fori_loop(0, _NPAIR, body, 0)
    drain(rows_a, gsem_a)
    drain(rows_b, gsem_b)


@jax.jit
def _gather(idx_t, table):
    run = pl.kernel(
        _gather_kernel,
        out_type=jax.ShapeDtypeStruct(
            (HIST_LEN, EMB // 8, _NW, 8, _BPW), jnp.float32),
        mesh=plsc.VectorSubcoreMesh(core_axis_name="c", subcore_axis_name="s"),
        scratch_types=[
            pltpu.VMEM((HIST_LEN, _BPW), jnp.int32),
            pltpu.VMEM((_BLK_ROWS, EMB), jnp.float32),
            pltpu.VMEM((_BLK_ROWS, EMB), jnp.float32),
            pltpu.VMEM((_K, EMB // 8, 1, 8, _BPW), jnp.float32),
            pltpu.VMEM((_K, EMB // 8, 1, 8, _BPW), jnp.float32),
            pltpu.SemaphoreType.DMA,
            pltpu.SemaphoreType.DMA,
            pltpu.SemaphoreType.DMA,
            pltpu.SemaphoreType.DMA,
        ],
        compiler_params=pltpu.CompilerParams(
            use_tc_tiling_on_sc=False, needs_layout_passes=False),
    )
    return run(idx_t, table)


def kernel(input, W):
    idx_t = jnp.transpose(input).astype(jnp.int32)
    out5 = _gather(idx_t, W)
    out = jnp.transpose(out5, (2, 4, 0, 1, 3)).reshape(BATCH, HIST_LEN, EMB)
    return jnp.expand_dims(out, -1)


# speedup vs baseline: 1.4124x; 1.0261x over previous
import functools

import jax
import jax.numpy as jnp
from jax import lax
from jax.experimental import pallas as pl
from jax.experimental.pallas import tpu as pltpu
from jax.experimental.pallas import tpu_sc as plsc

BATCH = 4096
HIST_LEN = 200
EMB = 32

_NC = 2
_NS = 16
_NW = _NC * _NS

_BPW = BATCH // _NW
_K = 4
_BLK_ROWS = _K * _BPW
_NBLK = HIST_LEN // _K
_NPAIR = _NBLK // 2


def _gather_kernel(idx_hbm, table_hbm, out_hbm,
                   idx_v, rows_a, rows_b, tbuf_a, tbuf_b,
                   gsem_a, gsem_b, osem_a, osem_b):
    wid = lax.axis_index("s") * _NC + lax.axis_index("c")
    col = wid * _BPW
    pltpu.sync_copy(idx_hbm.at[:, pl.ds(col, _BPW)], idx_v)

    lanes = jnp.arange(16, dtype=jnp.int32)

    def fire(rows, gsem, blk):
        for k in range(_K):
            pltpu.async_copy(
                table_hbm.at[idx_v.at[blk * _K + k]],
                rows.at[pl.ds(k * _BPW, _BPW)],
                gsem,
            )

    def drain(rows, gsem):
        pltpu.make_async_copy(table_hbm.at[pl.ds(0, _BLK_ROWS)], rows, gsem
                              ).wait()

    def transpose(rows, tbuf):
        @plsc.parallel_loop(0, _BLK_ROWS // 16, unroll=2)
        def tstep(g):
            r0 = g * 16
            k = g // 8
            off = (g % 8) * 16
            row_idx = lanes + r0
            for e0 in (0, 16):
                vs = [
                    plsc.load_gather(
                        rows,
                        [row_idx, jnp.full((16,), e0 + i, jnp.int32)])
                    for i in range(16)
                ]
                for i in range(16):
                    e = e0 + i
                    tbuf[k, e // 8, 0, e % 8, pl.ds(off, 16)] = vs[i]

    fire(rows_a, gsem_a, 0)
    fire(rows_b, gsem_b, 1)

    def body(p, carry):
        blk_a = 2 * p
        blk_b = 2 * p + 1
        drain(rows_a, gsem_a)
        transpose(rows_a, tbuf_a)
        out_a = pltpu.async_copy(
            tbuf_a,
            out_hbm.at[pl.ds(blk_a * _K, _K), :, pl.ds(wid, 1)],
            osem_a)
        fire(rows_a, gsem_a, lax.rem(blk_a + 2, _NBLK))
        drain(rows_b, gsem_b)
        transpose(rows_b, tbuf_b)
        out_b = pltpu.async_copy(
            tbuf_b,
            out_hbm.at[pl.ds(blk_b * _K, _K), :, pl.ds(wid, 1)],
            osem_b)
        fire(rows_b, gsem_b, lax.rem(blk_b + 2, _NBLK))
        out_a.wait()
        out_b.wait()
        return carry

    lax.fori_loop(0, _NPAIR, body, 0)
    drain(rows_a, gsem_a)
    drain(rows_b, gsem_b)


@jax.jit
def _gather(idx_t, table):
    run = pl.kernel(
        _gather_kernel,
        out_type=jax.ShapeDtypeStruct(
            (HIST_LEN, EMB // 8, _NW, 8, _BPW), jnp.float32),
        mesh=plsc.VectorSubcoreMesh(core_axis_name="c", subcore_axis_name="s"),
        scratch_types=[
            pltpu.VMEM((HIST_LEN, _BPW), jnp.int32),
            pltpu.VMEM((_BLK_ROWS, EMB), jnp.float32),
            pltpu.VMEM((_BLK_ROWS, EMB), jnp.float32),
            pltpu.VMEM((_K, EMB // 8, 1, 8, _BPW), jnp.float32),
            pltpu.VMEM((_K, EMB // 8, 1, 8, _BPW), jnp.float32),
            pltpu.SemaphoreType.DMA,
            pltpu.SemaphoreType.DMA,
            pltpu.SemaphoreType.DMA,
            pltpu.SemaphoreType.DMA,
        ],
        compiler_params=pltpu.CompilerParams(
            use_tc_tiling_on_sc=False, needs_layout_passes=False),
    )
    return run(idx_t, table)


def kernel(input, W):
    idx_t = jnp.transpose(input).astype(jnp.int32)
    out5 = _gather(idx_t, W)
    out = jnp.transpose(out5, (2, 4, 0, 1, 3)).reshape(BATCH, HIST_LEN, EMB)
    return jnp.expand_dims(out, -1)


# speedup vs baseline: 1.9754x; 1.3987x over previous
import functools

import jax
import jax.numpy as jnp
from jax import lax
from jax.experimental import pallas as pl
from jax.experimental.pallas import tpu as pltpu
from jax.experimental.pallas import tpu_sc as plsc

BATCH = 4096
HIST_LEN = 200
EMB = 32

_NC = 2
_NS = 16
_NW = _NC * _NS

_BPW = BATCH // _NW
_K = 4
_BLK_ROWS = _K * _BPW
_NBLK = HIST_LEN // _K
_NPAIR = _NBLK // 2
_TB = EMB * _BPW


def _gather_kernel(idx_hbm, table_hbm, out_hbm,
                   idx_v, rows_a, rows_b, tbuf_a, tbuf_b,
                   gsem_a, gsem_b, osem_a, osem_b):
    wid = lax.axis_index("s") * _NC + lax.axis_index("c")
    col = wid * _BPW
    pltpu.sync_copy(idx_hbm.at[:, pl.ds(col, _BPW)], idx_v)

    lanes = jnp.arange(16, dtype=jnp.int32)

    def fire(rows, gsem, blk):
        for k in range(_K):
            pltpu.async_copy(
                table_hbm.at[idx_v.at[blk * _K + k]],
                rows.at[pl.ds(k * _BPW, _BPW)],
                gsem,
            )

    def drain(rows, gsem):
        pltpu.make_async_copy(table_hbm.at[pl.ds(0, _BLK_ROWS)], rows, gsem
                              ).wait()

    def transpose(rows, tbuf):
        @plsc.parallel_loop(0, _BLK_ROWS // 16, unroll=2)
        def tstep(g):
            r0 = g * 16
            k = g // 8
            off = (g % 8) * 16
            row_idx = lanes + r0
            voff = lanes + (k * _TB + off)
            for e in range(EMB):
                col_idx = (lanes + e) & (EMB - 1)
                v = plsc.load_gather(rows, [row_idx, col_idx])
                plsc.store_scatter(tbuf, [(col_idx << 7) + voff], v)

    fire(rows_a, gsem_a, 0)
    fire(rows_b, gsem_b, 1)

    def fire_outs(tbuf, osem, blk):
        for k in range(_K):
            for eg in range(EMB // 8):
                pltpu.async_copy(
                    tbuf.at[pl.ds(k * _TB + eg * 1024, 1024)],
                    out_hbm.at[blk * _K + k, eg, wid],
                    osem,
                )

    def body(p, carry):
        blk_a = 2 * p
        blk_b = 2 * p + 1
        drain(rows_a, gsem_a)
        transpose(rows_a, tbuf_a)
        fire_outs(tbuf_a, osem_a, blk_a)
        fire(rows_a, gsem_a, lax.rem(blk_a + 2, _NBLK))
        drain(rows_b, gsem_b)
        transpose(rows_b, tbuf_b)
        fire_outs(tbuf_b, osem_b, blk_b)
        fire(rows_b, gsem_b, lax.rem(blk_b + 2, _NBLK))
        pltpu.make_async_copy(table_hbm.at[pl.ds(0, _BLK_ROWS)], tbuf_a,
                              osem_a).wait()
        pltpu.make_async_copy(table_hbm.at[pl.ds(0, _BLK_ROWS)], tbuf_b,
                              osem_b).wait()
        return carry

    lax.fori_loop(0, _NPAIR, body, 0)
    drain(rows_a, gsem_a)
    drain(rows_b, gsem_b)


@jax.jit
def _gather(idx_t, table):
    run = pl.kernel(
        _gather_kernel,
        out_type=jax.ShapeDtypeStruct(
            (HIST_LEN, EMB // 8, _NW, 8 * _BPW), jnp.float32),
        mesh=plsc.VectorSubcoreMesh(core_axis_name="c", subcore_axis_name="s"),
        scratch_types=[
            pltpu.VMEM((HIST_LEN, _BPW), jnp.int32),
            pltpu.VMEM((_BLK_ROWS, EMB), jnp.float32),
            pltpu.VMEM((_BLK_ROWS, EMB), jnp.float32),
            pltpu.VMEM((_K * _TB,), jnp.float32),
            pltpu.VMEM((_K * _TB,), jnp.float32),
            pltpu.SemaphoreType.DMA,
            pltpu.SemaphoreType.DMA,
            pltpu.SemaphoreType.DMA,
            pltpu.SemaphoreType.DMA,
        ],
        compiler_params=pltpu.CompilerParams(
            use_tc_tiling_on_sc=False, needs_layout_passes=False),
    )
    return run(idx_t, table)


def kernel(input, W):
    idx_t = jnp.transpose(input).astype(jnp.int32)
    out4 = _gather(idx_t, W)
    out5 = out4.reshape(HIST_LEN, EMB // 8, _NW, 8, _BPW)
    out = (jnp.transpose(out5, (2, 4, 0, 1, 3))
           .reshape(BATCH, HIST_LEN, EMB))
    return jnp.expand_dims(out, -1)
